# initial kernel scaffold (unmeasured)
import jax
import jax.numpy as jnp
from jax import lax
from jax.experimental import pallas as pl
from jax.experimental.pallas import tpu as pltpu

N_DEV = 8


def _gelu(y):
    c = 0.7978845608028654
    return 0.5 * y * (1.0 + jnp.tanh(c * (y + 0.044715 * y * y * y)))


def kernel(x, w_mat):
    m, _ = x.shape
    _, n = w_mat.shape
    cm = m // N_DEV

    def body(
        x_ref,
        w_ref,
        out_ref,
        acc_ref,
        rs_buf,
        rs_send_sems,
        rs_recv_sems,
        ag_send_sems,
        ag_recv_sems,
    ):
        d = lax.axis_index("i")
        right = lax.rem(d + 1, N_DEV)

        xb = x_ref[...].astype(jnp.bfloat16)
        wb = w_ref[...].astype(jnp.bfloat16)
        acc_ref[...] = jnp.dot(xb, wb, preferred_element_type=jnp.float32)

        for h in range(N_DEV - 1):
            send_i = lax.rem(d - h + N_DEV, N_DEV)
            recv_i = lax.rem(d - h - 1 + N_DEV, N_DEV)
            rdma = pltpu.make_async_remote_copy(
                src_ref=acc_ref.at[pl.ds(send_i * cm, cm), :],
                dst_ref=rs_buf.at[h],
                send_sem=rs_send_sems.at[h],
                recv_sem=rs_recv_sems.at[h],
                device_id=(right,),
                device_id_type=pl.DeviceIdType.MESH,
            )
            rdma.start()
            rdma.wait()
            acc_ref[pl.ds(recv_i * cm, cm), :] += rs_buf[h]

        red_i = lax.rem(d + 1, N_DEV)
        y = acc_ref[pl.ds(red_i * cm, cm), :]
        out_ref[pl.ds(red_i * cm, cm), :] = _gelu(y)

        for h in range(N_DEV - 1):
            send_i = lax.rem(d + 1 - h + N_DEV, N_DEV)
            rdma = pltpu.make_async_remote_copy(
                src_ref=out_ref.at[pl.ds(send_i * cm, cm), :],
                dst_ref=out_ref.at[pl.ds(send_i * cm, cm), :],
                send_sem=ag_send_sems.at[h],
                recv_sem=ag_recv_sems.at[h],
                device_id=(right,),
                device_id_type=pl.DeviceIdType.MESH,
            )
            rdma.start()
            rdma.wait()

    return pl.pallas_call(
        body,
        out_shape=jax.ShapeDtypeStruct((m, n), jnp.float32),
        in_specs=[
            pl.BlockSpec(memory_space=pltpu.VMEM),
            pl.BlockSpec(memory_space=pltpu.VMEM),
        ],
        out_specs=pl.BlockSpec(memory_space=pltpu.VMEM),
        scratch_shapes=[
            pltpu.VMEM((m, n), jnp.float32),
            pltpu.VMEM((N_DEV - 1, cm, n), jnp.float32),
            pltpu.SemaphoreType.DMA((N_DEV - 1,)),
            pltpu.SemaphoreType.DMA((N_DEV - 1,)),
            pltpu.SemaphoreType.DMA((N_DEV - 1,)),
            pltpu.SemaphoreType.DMA((N_DEV - 1,)),
        ],
        compiler_params=pltpu.CompilerParams(collective_id=0),
    )(x, w_mat)


# baseline (device time: 116510 ns/iter reference)
import jax
import jax.numpy as jnp
from jax import lax
from jax.experimental import pallas as pl
from jax.experimental.pallas import tpu as pltpu

N_DEV = 8


def _gelu(y):
    c = 0.7978845608028654
    return 0.5 * y * (1.0 + jnp.tanh(c * (y + 0.044715 * y * y * y)))


def kernel(x, w_mat):
    m, _ = x.shape
    _, n = w_mat.shape
    cm = m // N_DEV

    def body(
        x_ref,
        w_ref,
        out_ref,
        acc_ref,
        rs_buf,
        rs_send_sems,
        rs_recv_sems,
        ag_send_sems,
        ag_recv_sems,
    ):
        d = lax.axis_index("i")
        right = lax.rem(d + 1, N_DEV)

        xb = x_ref[...].astype(jnp.bfloat16)
        wb = w_ref[...].astype(jnp.bfloat16)
        acc_ref[...] = jnp.dot(xb, wb, preferred_element_type=jnp.float32)

        for h in range(N_DEV - 1):
            send_i = lax.rem(d - h + N_DEV, N_DEV)
            recv_i = lax.rem(d - h - 1 + N_DEV, N_DEV)
            rdma = pltpu.make_async_remote_copy(
                src_ref=acc_ref.at[pl.ds(send_i * cm, cm), :],
                dst_ref=rs_buf.at[h],
                send_sem=rs_send_sems.at[h],
                recv_sem=rs_recv_sems.at[h],
                device_id=(right,),
                device_id_type=pl.DeviceIdType.MESH,
            )
            rdma.start()
            rdma.wait()
            acc_ref[pl.ds(recv_i * cm, cm), :] += rs_buf[h]

        red_i = lax.rem(d + 1, N_DEV)
        y = acc_ref[pl.ds(red_i * cm, cm), :]
        out_ref[pl.ds(red_i * cm, cm), :] = _gelu(y)

        for h in range(N_DEV - 1):
            send_i = lax.rem(d + 1 - h + N_DEV, N_DEV)
            rdma = pltpu.make_async_remote_copy(
                src_ref=out_ref.at[pl.ds(send_i * cm, cm), :],
                dst_ref=out_ref.at[pl.ds(send_i * cm, cm), :],
                send_sem=ag_send_sems.at[h],
                recv_sem=ag_recv_sems.at[h],
                device_id=(right,),
                device_id_type=pl.DeviceIdType.MESH,
            )
            rdma.start()
            rdma.wait()

    return pl.pallas_call(
        body,
        out_shape=jax.ShapeDtypeStruct((m, n), jnp.float32),
        in_specs=[
            pl.BlockSpec(memory_space=pltpu.VMEM),
            pl.BlockSpec(memory_space=pltpu.VMEM),
        ],
        out_specs=pl.BlockSpec(memory_space=pltpu.VMEM),
        scratch_shapes=[
            pltpu.VMEM((m, n), jnp.float32),
            pltpu.VMEM((N_DEV - 1, cm, n), jnp.float32),
            pltpu.SemaphoreType.DMA((N_DEV - 1,)),
            pltpu.SemaphoreType.DMA((N_DEV - 1,)),
            pltpu.SemaphoreType.DMA((N_DEV - 1,)),
            pltpu.SemaphoreType.DMA((N_DEV - 1,)),
        ],
    )(x, w_mat)


# device time: 44619 ns/iter; 2.6112x vs baseline; 2.6112x over previous
import jax
import jax.numpy as jnp
from jax import lax
from jax.experimental import pallas as pl
from jax.experimental.pallas import tpu as pltpu

N_DEV = 8


def _gelu(y):
    c = 0.7978845608028654
    return 0.5 * y * (1.0 + jnp.tanh(c * (y + 0.044715 * y * y * y)))


def kernel(x, w_mat):
    m, _ = x.shape
    _, n = w_mat.shape
    cm = m // N_DEV

    def body(
        x_ref,
        w_ref,
        out_ref,
        acc_ref,
        pbf_ref,
        rs_buf,
        g_buf,
        ag_buf,
        rs_send_sems,
        rs_recv_sems,
        ag_send_sems,
        ag_recv_sems,
    ):
        d = lax.axis_index("i")

        xb = x_ref[...].astype(jnp.bfloat16)
        wb = w_ref[...].astype(jnp.bfloat16)
        acc_ref[...] = jnp.dot(xb, wb, preferred_element_type=jnp.float32)
        pbf_ref[...] = acc_ref[...].astype(jnp.bfloat16)

        rs_sends = []
        for t in range(1, N_DEV):
            tgt = lax.rem(d + t, N_DEV)
            slot = N_DEV - 1 - t
            rdma = pltpu.make_async_remote_copy(
                src_ref=pbf_ref.at[pl.ds(tgt * cm, cm), :],
                dst_ref=rs_buf.at[slot],
                send_sem=rs_send_sems.at[t - 1],
                recv_sem=rs_recv_sems.at[slot],
                device_id=(tgt,),
                device_id_type=pl.DeviceIdType.MESH,
            )
            rdma.start()
            rs_sends.append(rdma)

        dd = d * cm
        for slot in range(N_DEV - 1):
            recv = pltpu.make_async_remote_copy(
                src_ref=rs_buf.at[slot],
                dst_ref=rs_buf.at[slot],
                send_sem=rs_send_sems.at[0],
                recv_sem=rs_recv_sems.at[slot],
                device_id=(d,),
                device_id_type=pl.DeviceIdType.MESH,
            )
            recv.wait_recv()
            acc_ref[pl.ds(dd, cm), :] += rs_buf[slot].astype(jnp.float32)

        y = acc_ref[pl.ds(dd, cm), :]
        g = _gelu(y)
        out_ref[pl.ds(dd, cm), :] = g
        g_buf[...] = g.astype(jnp.bfloat16)

        ag_sends = []
        for t in range(1, N_DEV):
            tgt = lax.rem(d + t, N_DEV)
            slot = N_DEV - 1 - t
            rdma = pltpu.make_async_remote_copy(
                src_ref=g_buf,
                dst_ref=ag_buf.at[slot],
                send_sem=ag_send_sems.at[t - 1],
                recv_sem=ag_recv_sems.at[slot],
                device_id=(tgt,),
                device_id_type=pl.DeviceIdType.MESH,
            )
            rdma.start()
            ag_sends.append(rdma)

        for slot in range(N_DEV - 1):
            recv = pltpu.make_async_remote_copy(
                src_ref=ag_buf.at[slot],
                dst_ref=ag_buf.at[slot],
                send_sem=ag_send_sems.at[0],
                recv_sem=ag_recv_sems.at[slot],
                device_id=(d,),
                device_id_type=pl.DeviceIdType.MESH,
            )
            recv.wait_recv()
            ci = lax.rem(d + slot + 1, N_DEV)
            out_ref[pl.ds(ci * cm, cm), :] = ag_buf[slot].astype(jnp.float32)

        for rdma in rs_sends:
            rdma.wait_send()
        for rdma in ag_sends:
            rdma.wait_send()

    return pl.pallas_call(
        body,
        out_shape=jax.ShapeDtypeStruct((m, n), jnp.float32),
        in_specs=[
            pl.BlockSpec(memory_space=pltpu.VMEM),
            pl.BlockSpec(memory_space=pltpu.VMEM),
        ],
        out_specs=pl.BlockSpec(memory_space=pltpu.VMEM),
        scratch_shapes=[
            pltpu.VMEM((m, n), jnp.float32),
            pltpu.VMEM((m, n), jnp.bfloat16),
            pltpu.VMEM((N_DEV - 1, cm, n), jnp.bfloat16),
            pltpu.VMEM((cm, n), jnp.bfloat16),
            pltpu.VMEM((N_DEV - 1, cm, n), jnp.bfloat16),
            pltpu.SemaphoreType.DMA((N_DEV - 1,)),
            pltpu.SemaphoreType.DMA((N_DEV - 1,)),
            pltpu.SemaphoreType.DMA((N_DEV - 1,)),
            pltpu.SemaphoreType.DMA((N_DEV - 1,)),
        ],
    )(x, w_mat)
